# baseline (speedup 1.0000x reference)
import jax
import jax.numpy as jnp
from jax import lax
from jax.experimental import pallas as pl
from jax.experimental.pallas import tpu as pltpu
from jax.experimental.pallas import tpu_sc as plsc

_NUM_NODES = 1000000
_NUM_CHANNELS = 32
_BATCH = 16384

_INFO = plsc.get_sparse_core_info()
_NC = _INFO.num_cores
_NS = _INFO.num_subcores
_NW = _NC * _NS
_B_PER_W = _BATCH // _NW
_CHUNK = 128
_K = _B_PER_W // _CHUNK
_ROWS_PER_SUPER = 128 // _NUM_CHANNELS
_OUT_ROWS_W = _B_PER_W * _NUM_CHANNELS // 128


def _make_gather():
    mesh = plsc.VectorSubcoreMesh(core_axis_name="c", subcore_axis_name="s")

    @pl.kernel(
        mesh=mesh,
        out_type=jax.ShapeDtypeStruct((_BATCH * _NUM_CHANNELS // 128, 128),
                                      jnp.float32),
        scratch_types=[
            pltpu.VMEM((_K, _CHUNK), jnp.int32),
            pltpu.VMEM((_K, _CHUNK), jnp.int32),
            pltpu.VMEM((_B_PER_W, 128), jnp.float32),
            pltpu.VMEM((_OUT_ROWS_W, 128), jnp.float32),
            pltpu.SemaphoreType.DMA,
        ],
        compiler_params=pltpu.CompilerParams(needs_layout_passes=False),
    )
    def gather(idx_hbm, table_hbm, out_hbm, idx_v, ridx_v, rows_v, out_v, sem):
        wid = lax.axis_index("s") * _NC + lax.axis_index("c")
        pltpu.sync_copy(idx_hbm.at[wid], idx_v)
        for j in range(_K):
            for k in range(_CHUNK // 16):
                sl = pl.ds(k * 16, 16)
                ridx_v[j, sl] = idx_v[j, sl] >> 2
        copies = []
        for j in range(_K):
            copies.append(
                pltpu.async_copy(
                    table_hbm.at[ridx_v.at[j]],
                    rows_v.at[pl.ds(j * _CHUNK, _CHUNK)],
                    sem,
                )
            )
        for c in copies:
            c.wait()

        iota = lax.iota(jnp.int32, 16)

        def select_row(i, _):
            jv = jnp.full((16,), i >> 7, jnp.int32)
            kv = jnp.full((16,), i & 127, jnp.int32)
            idxv = plsc.load_gather(idx_v, [jv, kv])
            cv = ((idxv & (_ROWS_PER_SUPER - 1)) << 5) + iota
            iv = jnp.full((16,), i, jnp.int32)
            v0 = plsc.load_gather(rows_v, [iv, cv])
            v1 = plsc.load_gather(rows_v, [iv, cv + 16])
            r = i >> 2
            c0 = (i & 3) << 5
            out_v[r, pl.ds(c0, 16)] = v0
            out_v[r, pl.ds(c0 + 16, 16)] = v1
            return _

        lax.fori_loop(0, _B_PER_W, select_row, None, unroll=4)

        pltpu.sync_copy(out_v, out_hbm.at[pl.ds(wid * _OUT_ROWS_W, _OUT_ROWS_W)])

    return gather


_gather = _make_gather()


@jax.jit
def kernel(x_indices, embedding_weight):
    idx = jnp.reshape(x_indices.astype(jnp.int32), (_NW, _K, _CHUNK))
    table = jnp.reshape(embedding_weight, (_NUM_NODES // _ROWS_PER_SUPER, 128))
    out = _gather(idx, table)
    return jnp.reshape(out, (_BATCH, _NUM_CHANNELS))


# speedup vs baseline: 3.9157x; 3.9157x over previous
import jax
import jax.numpy as jnp
from jax import lax
from jax.experimental import pallas as pl
from jax.experimental.pallas import tpu as pltpu
from jax.experimental.pallas import tpu_sc as plsc

_NUM_NODES = 1000000
_NUM_CHANNELS = 32
_BATCH = 16384

_INFO = plsc.get_sparse_core_info()
_NC = _INFO.num_cores
_NS = _INFO.num_subcores
_NW = _NC * _NS
_B_PER_W = _BATCH // _NW
_G = 16
_NGROUPS = _B_PER_W // _G


def _make_gather():
    mesh = plsc.VectorSubcoreMesh(core_axis_name="c", subcore_axis_name="s")

    @pl.kernel(
        mesh=mesh,
        out_type=jax.ShapeDtypeStruct((_NUM_CHANNELS, _BATCH), jnp.float32),
        scratch_types=[
            pltpu.VMEM((_B_PER_W,), jnp.int32),
            pltpu.VMEM((_G, _NUM_CHANNELS, 128), jnp.float32),
            pltpu.VMEM((_NUM_CHANNELS, _B_PER_W), jnp.float32),
            pltpu.SemaphoreType.DMA,
        ],
        compiler_params=pltpu.CompilerParams(needs_layout_passes=False),
    )
    def gather(idx_hbm, table_hbm, out_hbm, idx_v, ring, out_v, sem):
        wid = lax.axis_index("s") * _NC + lax.axis_index("c")
        base = wid * _B_PER_W
        pltpu.sync_copy(idx_hbm.at[pl.ds(base, _B_PER_W)], idx_v)
        iota = lax.iota(jnp.int32, 16)

        def group(g, _):
            nv = idx_v[pl.ds(g * _G, _G)]
            copies = []
            for j in range(_G):
                n = jnp.sum(jnp.where(iota == j, nv, 0))
                gbase = pl.multiple_of((n >> 7) * 128, 128)
                copies.append(
                    pltpu.async_copy(
                        table_hbm.at[:, pl.ds(gbase, 128)],
                        ring.at[j],
                        sem,
                    )
                )
            for cp in copies:
                cp.wait()
            colv = idx_v[pl.ds(g * _G, _G)] & 127
            for c in range(_NUM_CHANNELS):
                cv = jnp.full((16,), c, jnp.int32)
                v = plsc.load_gather(ring, [iota, cv, colv])
                out_v[c, pl.ds(g * _G, _G)] = v
            return _

        lax.fori_loop(0, _NGROUPS, group, None)
        pltpu.sync_copy(out_v, out_hbm.at[:, pl.ds(base, _B_PER_W)])

    return gather


_gather = _make_gather()


@jax.jit
def kernel(x_indices, embedding_weight):
    idx = x_indices.astype(jnp.int32)
    table_t = embedding_weight.T
    out_t = _gather(idx, table_t)
    return out_t.T
